# baseline (speedup 1.0000x reference)
import functools

import jax
import jax.numpy as jnp
from jax import lax
from jax.experimental import pallas as pl
from jax.experimental.pallas import tpu as pltpu
from jax.experimental.pallas import tpu_sc as plsc

B, S, CIN = 4096, 100, 128
COUT = 64
NROWS = B * S
NW = 32
ROWS_PER_W = NROWS // NW
R = 256
NCHUNK = ROWS_PER_W // R
GROUPS = R * COUT // 16
UNROLL = 8

_mesh = plsc.VectorSubcoreMesh(core_axis_name="c", subcore_axis_name="s")


@functools.partial(
    pl.kernel,
    mesh=_mesh,
    out_type=jax.ShapeDtypeStruct((NROWS * COUT,), jnp.float32),
    scratch_types=[
        pltpu.VMEM((R * CIN,), jnp.float32),
        pltpu.VMEM((R * COUT,), jnp.float32),
    ],
    compiler_params=pltpu.CompilerParams(needs_layout_passes=False),
)
def _deinterleave_sc(x_hbm, out_hbm, in_v, out_v):
    wid = lax.axis_index("s") * 2 + lax.axis_index("c")
    row0 = wid * ROWS_PER_W
    it2 = lax.iota(jnp.int32, 16) * 2

    def chunk_body(c, carry):
        base = row0 + c * R
        pltpu.sync_copy(x_hbm.at[pl.ds(base * CIN, R * CIN)], in_v)

        def g_body(g, carry2):
            for u in range(UNROLL):
                gg = g * UNROLL + u
                vals = plsc.load_gather(in_v, [it2 + gg * 32])
                out_v[pl.ds(gg * 16, 16)] = vals
            return carry2

        lax.fori_loop(0, GROUPS // UNROLL, g_body, 0)
        pltpu.sync_copy(out_v, out_hbm.at[pl.ds(base * COUT, R * COUT)])
        return carry

    lax.fori_loop(0, NCHUNK, chunk_body, 0)


def kernel(x):
    out_flat = _deinterleave_sc(x.reshape(-1))
    return out_flat.reshape(B, S, COUT)


# speedup vs baseline: 1.9650x; 1.9650x over previous
import functools

import jax
import jax.numpy as jnp
from jax import lax
from jax.experimental import pallas as pl
from jax.experimental.pallas import tpu as pltpu
from jax.experimental.pallas import tpu_sc as plsc

B, S, CIN = 4096, 100, 128
COUT = 64
NW = 32
NB = 256
NBLK = B // NB
NCHUNK_TOTAL = S * NBLK
CPW = NCHUNK_TOTAL // NW

WIN = NB * CIN
WOUT = NB * COUT

S_STRIDE = COUT * B
TR_STRIDE = 8 * B
RUN = 2 * 1024

_mesh = plsc.VectorSubcoreMesh(core_axis_name="c", subcore_axis_name="s")


@functools.partial(
    pl.kernel,
    mesh=_mesh,
    out_type=jax.ShapeDtypeStruct((S * COUT * B,), jnp.float32),
    scratch_types=[
        pltpu.VMEM((WIN,), jnp.float32),
        pltpu.VMEM((WIN,), jnp.float32),
        pltpu.VMEM((WOUT,), jnp.float32),
        pltpu.VMEM((WOUT,), jnp.float32),
        pltpu.SemaphoreType.DMA,
        pltpu.SemaphoreType.DMA,
        pltpu.SemaphoreType.DMA,
        pltpu.SemaphoreType.DMA,
    ],
    compiler_params=pltpu.CompilerParams(needs_layout_passes=False),
)
def _deinterleave_sc(x_hbm, out_hbm, in_v0, in_v1, out_v0, out_v1,
                     si0, si1, so0, so1):
    wid = lax.axis_index("s") * 2 + lax.axis_index("c")
    c0 = wid * CPW
    it128 = lax.iota(jnp.int32, 16) * CIN

    in_bufs = (in_v0, in_v1)
    out_bufs = (out_v0, out_v1)
    in_sems = (si0, si1)
    out_sems = (so0, so1)

    def chunk_coords(c):
        a = c0 + c
        s = a // NBLK
        blk = a - s * NBLK
        return s, blk

    def in_slice(c):
        s, blk = chunk_coords(c)
        return x_hbm.at[pl.ds((s * B + blk * NB) * CIN, WIN)]

    def out_run(c, tr):
        s, blk = chunk_coords(c)
        start = s * S_STRIDE + tr * TR_STRIDE + blk * RUN
        return out_hbm.at[pl.ds(start, RUN)]

    def start_in(c, b):
        pltpu.async_copy(in_slice(c), in_bufs[b], in_sems[b])

    def wait_in(c, b):
        pltpu.make_async_copy(in_slice(c), in_bufs[b], in_sems[b]).wait()

    def start_out(c, b):
        for tr in range(8):
            pltpu.async_copy(out_bufs[b].at[pl.ds(tr * RUN, RUN)],
                             out_run(c, tr), out_sems[b])

    def wait_out(c, b):
        for tr in range(8):
            pltpu.make_async_copy(out_bufs[b].at[pl.ds(tr * RUN, RUN)],
                                  out_run(c, tr), out_sems[b]).wait()

    def compute(in_b, out_b):
        @plsc.parallel_loop(0, 8, unroll=2)
        def tr_body(tr):
            vtr = it128 + tr * 16
            vals = [plsc.load_gather(
                        in_b, [vtr + ((tc2 * 128 + bg * 16) * CIN + 2 * jj)])
                    for tc2 in range(2) for jj in range(8) for bg in range(8)]
            i = 0
            for tc2 in range(2):
                for jj in range(8):
                    for bg in range(8):
                        out_b[pl.ds(tr * RUN + tc2 * 1024 + jj * 128 + bg * 16,
                                    16)] = vals[i]
                        i += 1

    start_in(0, 0)
    start_in(1, 1)

    def outer(i, carry):
        for b in range(2):
            c = i * 2 + b
            wait_in(c, b)
            @pl.when(c >= 2)
            def _wait_out():
                wait_out(c, b)
            compute(in_bufs[b], out_bufs[b])
            start_out(c, b)
            @pl.when(c + 2 < CPW)
            def _next_in():
                start_in(c + 2, b)
        return carry

    lax.fori_loop(0, CPW // 2, outer, 0)

    wait_out(CPW - 2, 0)
    wait_out(CPW - 1, 1)


def kernel(x):
    x_lin = jnp.transpose(x, (1, 0, 2)).reshape(-1)
    o = _deinterleave_sc(x_lin)
    o5 = o.reshape(S, 8, B // 128, 8, 128)
    return o5.transpose(2, 4, 0, 1, 3).reshape(B, S, COUT)


# speedup vs baseline: 2.6670x; 1.3572x over previous
import functools

import jax
import jax.numpy as jnp
from jax import lax
from jax.experimental import pallas as pl
from jax.experimental.pallas import tpu as pltpu
from jax.experimental.pallas import tpu_sc as plsc

B, S, CIN = 4096, 100, 128
COUT = 64
NROWS = B * S
NW = 32
ROWS_PER_W = NROWS // NW
R = 256
NCHUNK = ROWS_PER_W // R
UNROLL = 4

WIN = R * CIN
WOUT = R * COUT

_mesh = plsc.VectorSubcoreMesh(core_axis_name="c", subcore_axis_name="s")


@functools.partial(
    pl.kernel,
    mesh=_mesh,
    out_type=jax.ShapeDtypeStruct((NROWS * COUT,), jnp.float32),
    scratch_types=[
        pltpu.VMEM((WIN,), jnp.float32),
        pltpu.VMEM((WIN,), jnp.float32),
        pltpu.VMEM((WOUT,), jnp.float32),
        pltpu.VMEM((WOUT,), jnp.float32),
        pltpu.SemaphoreType.DMA,
        pltpu.SemaphoreType.DMA,
        pltpu.SemaphoreType.DMA,
        pltpu.SemaphoreType.DMA,
    ],
    compiler_params=pltpu.CompilerParams(needs_layout_passes=False),
)
def _deinterleave_sc(x_hbm, out_hbm, in_v0, in_v1, out_v0, out_v1,
                     si0, si1, so0, so1):
    wid = lax.axis_index("s") * 2 + lax.axis_index("c")
    row0 = wid * ROWS_PER_W
    it2 = lax.iota(jnp.int32, 16) * 2

    in_bufs = (in_v0, in_v1)
    out_bufs = (out_v0, out_v1)
    in_sems = (si0, si1)
    out_sems = (so0, so1)

    def in_slice(c):
        return x_hbm.at[pl.ds((row0 + c * R) * CIN, WIN)]

    def out_slice(c):
        return out_hbm.at[pl.ds((row0 + c * R) * COUT, WOUT)]

    def compute(in_b, out_b):
        @plsc.parallel_loop(0, WOUT // 128, unroll=2)
        def g_body(gh):
            base = gh * 256
            vals = [plsc.load_gather(in_b, [it2 + (base + k * 32)])
                    for k in range(8)]
            for k in range(8):
                out_b[pl.ds(gh * 128 + k * 16, 16)] = vals[k]

    pltpu.async_copy(in_slice(0), in_bufs[0], in_sems[0])
    pltpu.async_copy(in_slice(1), in_bufs[1], in_sems[1])

    def outer(i, carry):
        for b in range(2):
            c = i * 2 + b
            pltpu.make_async_copy(in_slice(c), in_bufs[b], in_sems[b]).wait()
            @pl.when(c >= 2)
            def _wait_out():
                pltpu.make_async_copy(out_bufs[b], out_slice(c),
                                      out_sems[b]).wait()
            compute(in_bufs[b], out_bufs[b])
            pltpu.async_copy(out_bufs[b], out_slice(c), out_sems[b])
            @pl.when(c + 2 < NCHUNK)
            def _next_in():
                pltpu.async_copy(in_slice(c + 2), in_bufs[b], in_sems[b])
        return carry

    lax.fori_loop(0, NCHUNK // 2, outer, 0)

    pltpu.make_async_copy(out_bufs[0], out_slice(NCHUNK - 2), out_sems[0]).wait()
    pltpu.make_async_copy(out_bufs[1], out_slice(NCHUNK - 1), out_sems[1]).wait()


def kernel(x):
    x_lin = jnp.transpose(x, (1, 0, 2)).reshape(-1)
    o = _deinterleave_sc(x_lin)
    return o.reshape(S, B, COUT).transpose(1, 0, 2)


# speedup vs baseline: 7.1920x; 2.6967x over previous
import functools

import jax
import jax.numpy as jnp
from jax import lax
from jax.experimental import pallas as pl
from jax.experimental.pallas import tpu as pltpu
from jax.experimental.pallas import tpu_sc as plsc

B, S, CIN = 4096, 100, 128
COUT = 64
NW = 32
NB = 256
NBLK = B // NB
NCHUNK_TOTAL = S * NBLK
CPW = NCHUNK_TOTAL // NW

WIN = NB * CIN
WOUT = NB * COUT
MIDSTRIDE = 69
WMID = NB * MIDSTRIDE

S_STRIDE = COUT * B
TR_STRIDE = 8 * B
RUN = 2 * 1024

_mesh = plsc.VectorSubcoreMesh(core_axis_name="c", subcore_axis_name="s")


@functools.partial(
    pl.kernel,
    mesh=_mesh,
    out_type=jax.ShapeDtypeStruct((S * COUT * B,), jnp.float32),
    scratch_types=[
        pltpu.VMEM((WIN,), jnp.float32),
        pltpu.VMEM((WIN,), jnp.float32),
        pltpu.VMEM((WMID,), jnp.float32),
        pltpu.VMEM((WOUT,), jnp.float32),
        pltpu.VMEM((WOUT,), jnp.float32),
        pltpu.SemaphoreType.DMA,
        pltpu.SemaphoreType.DMA,
        pltpu.SemaphoreType.DMA,
        pltpu.SemaphoreType.DMA,
    ],
    compiler_params=pltpu.CompilerParams(needs_layout_passes=False),
)
def _deinterleave_sc(x_hbm, out_hbm, in_v0, in_v1, mid_v, out_v0, out_v1,
                     si0, si1, so0, so1):
    wid = lax.axis_index("s") * 2 + lax.axis_index("c")
    c0 = wid * CPW
    it2 = lax.iota(jnp.int32, 16) * 2
    it69 = lax.iota(jnp.int32, 16) * MIDSTRIDE

    in_bufs = (in_v0, in_v1)
    out_bufs = (out_v0, out_v1)
    in_sems = (si0, si1)
    out_sems = (so0, so1)

    def chunk_coords(c):
        a = c0 + c
        s = a // NBLK
        blk = a - s * NBLK
        return s, blk

    def in_slice(c):
        s, blk = chunk_coords(c)
        return x_hbm.at[pl.ds((s * B + blk * NB) * CIN, WIN)]

    def out_run(c, tr):
        s, blk = chunk_coords(c)
        start = s * S_STRIDE + tr * TR_STRIDE + blk * RUN
        return out_hbm.at[pl.ds(start, RUN)]

    def start_in(c, b):
        pltpu.async_copy(in_slice(c), in_bufs[b], in_sems[b])

    def wait_in(c, b):
        pltpu.make_async_copy(in_slice(c), in_bufs[b], in_sems[b]).wait()

    def start_out(c, b):
        for tr in range(8):
            pltpu.async_copy(out_bufs[b].at[pl.ds(tr * RUN, RUN)],
                             out_run(c, tr), out_sems[b])

    def wait_out(c, b):
        for tr in range(8):
            pltpu.make_async_copy(out_bufs[b].at[pl.ds(tr * RUN, RUN)],
                                  out_run(c, tr), out_sems[b]).wait()

    def compute(in_b, out_b):
        @plsc.parallel_loop(0, NB // 2, unroll=2)
        def a_body(r2):
            ibase = r2 * (2 * CIN)
            obase = r2 * (2 * MIDSTRIDE)
            vals = [plsc.load_gather(in_b, [it2 + (ibase + k * 32)])
                    for k in range(8)]
            for k in range(8):
                u, q = divmod(k, 4)
                mid_v[pl.ds(obase + u * MIDSTRIDE + q * 16, 16)] = vals[k]

        @plsc.parallel_loop(0, 128, unroll=2)
        def b_body(i):
            hi = i // 8
            tr = i - hi * 8
            tc2 = hi // 8
            bg = hi - tc2 * 8
            in_base = hi * (16 * MIDSTRIDE) + tr * 8
            out_base = tr * RUN + tc2 * 1024 + bg * 16
            vals = [plsc.load_gather(mid_v, [it69 + (in_base + jj)])
                    for jj in range(8)]
            for jj in range(8):
                out_b[pl.ds(out_base + jj * 128, 16)] = vals[jj]

    start_in(0, 0)
    start_in(1, 1)

    def outer(i, carry):
        for b in range(2):
            c = i * 2 + b
            wait_in(c, b)
            @pl.when(c >= 2)
            def _wait_out():
                wait_out(c, b)
            compute(in_bufs[b], out_bufs[b])
            start_out(c, b)
            @pl.when(c + 2 < CPW)
            def _next_in():
                start_in(c + 2, b)
        return carry

    lax.fori_loop(0, CPW // 2, outer, 0)

    wait_out(CPW - 2, 0)
    wait_out(CPW - 1, 1)


def kernel(x):
    x_lin = jnp.transpose(x, (1, 0, 2)).reshape(-1)
    o = _deinterleave_sc(x_lin)
    o5 = o.reshape(S, 8, B // 128, 8, 128)
    return o5.transpose(2, 4, 0, 1, 3).reshape(B, S, COUT)
